# baseline (speedup 1.0000x reference)
import functools

import jax
import jax.numpy as jnp
from jax import lax
from jax.experimental import pallas as pl
from jax.experimental.pallas import tpu as pltpu
from jax.experimental.pallas import tpu_sc as plsc

_VOCAB = 30
_D = 2048
_B = 4 * 8192
_SPLIT = 14336


def _sc_embed(tokens_flat, table, n):
  info = plsc.get_sparse_core_info()
  nc, ns = info.num_cores, info.num_subcores
  nw = nc * ns
  bpw = n // nw
  mesh = plsc.VectorSubcoreMesh(core_axis_name="c", subcore_axis_name="s")

  @functools.partial(
      pl.kernel,
      mesh=mesh,
      out_type=jax.ShapeDtypeStruct((n, _D), jnp.float32),
      scratch_types=[
          pltpu.VMEM((_VOCAB, _D), jnp.float32),
          pltpu.VMEM((bpw,), jnp.int32),
          pltpu.SemaphoreType.DMA,
          pltpu.SemaphoreType.DMA,
      ],
  )
  def k(tokens_hbm, table_hbm, out_hbm, table_v, idx_v, sem, sem_in):
    wid = lax.axis_index("s") * nc + lax.axis_index("c")
    base = wid * bpw
    tab_cp = pltpu.make_async_copy(table_hbm, table_v, sem_in)
    tab_cp.start()
    idx_cp = pltpu.make_async_copy(tokens_hbm.at[pl.ds(base, bpw)], idx_v,
                                   sem_in)
    idx_cp.start()
    idx_cp.wait()
    tab_cp.wait()

    def issue(g, carry):
      vec = idx_v[pl.ds(g * 16, 16)]
      for l in range(16):
        tok = vec[l]
        pltpu.make_async_copy(
            table_v.at[pl.ds(tok, 1)],
            out_hbm.at[pl.ds(base + g * 16 + l, 1)],
            sem,
        ).start()
      return carry

    lax.fori_loop(0, bpw // 16, issue, 0)

    pltpu.make_async_copy(
        out_hbm.at[pl.ds(base, bpw)],
        out_hbm.at[pl.ds(base, bpw)],
        sem,
    ).wait()

  return k(tokens_flat, table)


def _tc_embed(tokens_flat, table_pad, n):
  blk = 1024
  grid = (n // blk,)

  def body(tok_ref, tab_ref, out_ref):
    toks = tok_ref[0]
    oh_t = (lax.broadcasted_iota(jnp.int32, (32, blk), 0) == toks).astype(
        jnp.float32)
    out_ref[...] = lax.dot_general(
        oh_t, tab_ref[...], (((0,), (0,)), ((), ())),
        preferred_element_type=jnp.float32)

  return pl.pallas_call(
      body,
      grid=grid,
      in_specs=[
          pl.BlockSpec((1, 1, blk), lambda i: (i, 0, 0)),
          pl.BlockSpec((32, _D), lambda i: (0, 0)),
      ],
      out_specs=pl.BlockSpec((blk, _D), lambda i: (i, 0)),
      out_shape=jax.ShapeDtypeStruct((n, _D), jnp.float32),
  )(tokens_flat.reshape(n // blk, 1, blk), table_pad)


def kernel(sequence_tokens, sequence_embed_weight):
  b, s = sequence_tokens.shape
  tokens_flat = sequence_tokens.reshape(b * s)
  table_pad = jnp.pad(sequence_embed_weight, ((0, 32 - _VOCAB), (0, 0)))
  out_sc = _sc_embed(tokens_flat[:_SPLIT], sequence_embed_weight, _SPLIT)
  out_tc = _tc_embed(tokens_flat[_SPLIT:], table_pad, _B - _SPLIT)
  out = jnp.concatenate([out_sc, out_tc], axis=0)
  return out.reshape(b, s, _D)


# speedup vs baseline: 2.4154x; 2.4154x over previous
import functools

import jax
import jax.numpy as jnp
from jax import lax
from jax.experimental import pallas as pl
from jax.experimental.pallas import tpu as pltpu
from jax.experimental.pallas import tpu_sc as plsc

_VOCAB = 30
_D = 2048
_B = 4 * 8192
_SPLIT = 16384


def _sc_embed(tokens_flat, table, n):
  info = plsc.get_sparse_core_info()
  nc, ns = info.num_cores, info.num_subcores
  nw = nc * ns
  bpw = n // nw
  mesh = plsc.VectorSubcoreMesh(core_axis_name="c", subcore_axis_name="s")

  @functools.partial(
      pl.kernel,
      mesh=mesh,
      out_type=jax.ShapeDtypeStruct((_B, _D), jnp.float32),
      scratch_types=[
          pltpu.VMEM((_VOCAB, _D), jnp.float32),
          pltpu.VMEM((bpw,), jnp.int32),
          pltpu.SemaphoreType.DMA,
          pltpu.SemaphoreType.DMA,
      ],
  )
  def k(tokens_hbm, table_hbm, out_hbm, table_v, idx_v, sem, sem_in):
    wid = lax.axis_index("s") * nc + lax.axis_index("c")
    base = wid * bpw
    tab_cp = pltpu.make_async_copy(table_hbm, table_v, sem_in)
    tab_cp.start()
    idx_cp = pltpu.make_async_copy(tokens_hbm.at[pl.ds(base, bpw)], idx_v,
                                   sem_in)
    idx_cp.start()
    idx_cp.wait()
    tab_cp.wait()

    def issue(g, carry):
      vec = idx_v[pl.ds(g * 16, 16)]
      for l in range(16):
        tok = vec[l]
        pltpu.make_async_copy(
            table_v.at[pl.ds(tok, 1)],
            out_hbm.at[pl.ds(base + g * 16 + l, 1)],
            sem,
        ).start()
      return carry

    lax.fori_loop(0, bpw // 16, issue, 0)

    pltpu.make_async_copy(
        out_hbm.at[pl.ds(base, bpw)],
        out_hbm.at[pl.ds(base, bpw)],
        sem,
    ).wait()

  return k(tokens_flat, table)


def _tc_embed_into(tokens_flat, table_pad, partial_out, start):
  blk = 1024
  n = _B - start
  grid = (n // blk,)
  first = start // blk

  def body(tok_ref, tab_ref, alias_ref, out_ref):
    del alias_ref
    toks = tok_ref[0]
    oh_t = (lax.broadcasted_iota(jnp.int32, (32, blk), 0) == toks).astype(
        jnp.float32)
    out_ref[...] = lax.dot_general(
        oh_t, tab_ref[...], (((0,), (0,)), ((), ())),
        preferred_element_type=jnp.float32)

  return pl.pallas_call(
      body,
      grid=grid,
      in_specs=[
          pl.BlockSpec((1, 1, blk), lambda i: (i, 0, 0)),
          pl.BlockSpec((32, _D), lambda i: (0, 0)),
          pl.BlockSpec(memory_space=pl.MemorySpace.ANY),
      ],
      out_specs=pl.BlockSpec((blk, _D), lambda i: (i + first, 0)),
      out_shape=jax.ShapeDtypeStruct((_B, _D), jnp.float32),
      input_output_aliases={2: 0},
  )(tokens_flat.reshape(n // blk, 1, blk), table_pad, partial_out)


def kernel(sequence_tokens, sequence_embed_weight):
  b, s = sequence_tokens.shape
  tokens_flat = sequence_tokens.reshape(b * s)
  table_pad = jnp.pad(sequence_embed_weight, ((0, 32 - _VOCAB), (0, 0)))
  out_sc = _sc_embed(tokens_flat[:_SPLIT], sequence_embed_weight, _SPLIT)
  out = _tc_embed_into(tokens_flat[_SPLIT:], table_pad, out_sc, _SPLIT)
  return out.reshape(b, s, _D)


# speedup vs baseline: 2.5054x; 1.0373x over previous
import functools

import jax
import jax.numpy as jnp
from jax import lax
from jax.experimental import pallas as pl
from jax.experimental.pallas import tpu as pltpu
from jax.experimental.pallas import tpu_sc as plsc

_VOCAB = 30
_D = 2048
_B = 4 * 8192


def _sc_embed(tokens_flat, table):
  info = plsc.get_sparse_core_info()
  nc, ns = info.num_cores, info.num_subcores
  nw = nc * ns
  bpw = _B // nw
  mesh = plsc.VectorSubcoreMesh(core_axis_name="c", subcore_axis_name="s")

  @functools.partial(
      pl.kernel,
      mesh=mesh,
      out_type=jax.ShapeDtypeStruct((_B, _D), jnp.float32),
      scratch_types=[
          pltpu.VMEM((_VOCAB, _D), jnp.float32),
          pltpu.VMEM((bpw,), jnp.int32),
          pltpu.SemaphoreType.DMA,
      ],
  )
  def k(tokens_hbm, table_hbm, out_hbm, table_v, idx_s, sem):
    wid = lax.axis_index("s") * nc + lax.axis_index("c")
    base = wid * bpw
    pltpu.sync_copy(table_hbm, table_v)
    pltpu.sync_copy(tokens_hbm.at[pl.ds(base, bpw)], idx_s)

    def issue(g, carry):
      vec = idx_s[pl.ds(g * 16, 16)]
      for l in range(16):
        tok = vec[l]
        pltpu.make_async_copy(
            table_v.at[pl.ds(tok, 1)],
            out_hbm.at[pl.ds(base + g * 16 + l, 1)],
            sem,
        ).start()
      return carry

    lax.fori_loop(0, bpw // 16, issue, 0)

    pltpu.make_async_copy(
        out_hbm.at[pl.ds(base, bpw)],
        out_hbm.at[pl.ds(base, bpw)],
        sem,
    ).wait()

  return k(tokens_flat, table)


def kernel(sequence_tokens, sequence_embed_weight):
  b, s = sequence_tokens.shape
  out = _sc_embed(sequence_tokens.reshape(b * s), sequence_embed_weight)
  return out.reshape(b, s, _D)
